# baseline (speedup 1.0000x reference)
import dataclasses
import functools

import jax
import jax.numpy as jnp
from jax.experimental import pallas as pl
from jax.experimental.pallas import tpu as pltpu
from jax.experimental.pallas import tpu_sc as plsc

_C = 10
_LANES = 16
_CH = 2048


def kernel(mask):
    B, _, H, W = mask.shape
    P = H * W
    mask_flat = mask.reshape(B, P)

    mesh = plsc.VectorSubcoreMesh(core_axis_name="core",
                                  subcore_axis_name="subcore")
    cp = pltpu.CompilerParams(use_tc_tiling_on_sc=True)
    if "needs_layout_passes" in pltpu.CompilerParams.__dataclass_fields__:
        cp = dataclasses.replace(cp, needs_layout_passes=False)

    @functools.partial(
        pl.kernel,
        out_type=jax.ShapeDtypeStruct((B, _C, P), jnp.int32),
        mesh=mesh,
        compiler_params=cp,
    )
    def run(m_hbm, o_hbm):
        def body(m_vmem, o_vmem):
            @pl.loop(0, _CH, step=_LANES)
            def _(j):
                v = m_vmem[0, pl.ds(j, _LANES)]
                for c in range(_C):
                    o_vmem[0, c, pl.ds(j, _LANES)] = (v == c).astype(jnp.int32)

        pltpu.emit_pipeline(
            body,
            grid=(B, P // _CH),
            in_specs=[pl.BlockSpec((1, _CH), lambda b, i: (b, i))],
            out_specs=[pl.BlockSpec((1, _C, _CH), lambda b, i: (b, 0, i))],
            core_axis_name=("core", "subcore"),
            dimension_semantics=(pltpu.PARALLEL, pltpu.PARALLEL),
        )(m_hbm, o_hbm)

    out = run(mask_flat)
    return out.reshape(B, _C, H, W)


# speedup vs baseline: 3.6512x; 3.6512x over previous
import dataclasses
import functools

import jax
import jax.numpy as jnp
from jax.experimental import pallas as pl
from jax.experimental.pallas import tpu as pltpu
from jax.experimental.pallas import tpu_sc as plsc

_C = 10
_LANES = 16
_RB = 8


def kernel(mask):
    B, _, H, W = mask.shape

    mesh = plsc.VectorSubcoreMesh(core_axis_name="core",
                                  subcore_axis_name="subcore")
    cp = pltpu.CompilerParams(use_tc_tiling_on_sc=True)
    if "needs_layout_passes" in pltpu.CompilerParams.__dataclass_fields__:
        cp = dataclasses.replace(cp, needs_layout_passes=False)

    @functools.partial(
        pl.kernel,
        out_type=jax.ShapeDtypeStruct((B, _C, H, W), jnp.int32),
        mesh=mesh,
        compiler_params=cp,
    )
    def run(m_hbm, o_hbm):
        def body(m_vmem, o_vmem):
            @pl.loop(0, _RB)
            def _(r):
                @pl.loop(0, W, step=_LANES)
                def _(j):
                    v = m_vmem[0, 0, r, pl.ds(j, _LANES)]
                    for c in range(_C):
                        o_vmem[0, c, r, pl.ds(j, _LANES)] = (
                            v == c).astype(jnp.int32)

        pltpu.emit_pipeline(
            body,
            grid=(B, H // _RB),
            in_specs=[pl.BlockSpec((1, 1, _RB, W), lambda b, i: (b, 0, i, 0))],
            out_specs=[pl.BlockSpec((1, _C, _RB, W),
                                    lambda b, i: (b, 0, i, 0))],
            core_axis_name=("core", "subcore"),
            dimension_semantics=(pltpu.PARALLEL, pltpu.PARALLEL),
        )(m_hbm, o_hbm)

    return run(mask)
